# Initial kernel scaffold; baseline (speedup 1.0000x reference)
import jax
import jax.numpy as jnp
from jax.experimental import pallas as pl


def kernel(x):
    raise NotImplementedError("write your pallas kernel here")


# speedup vs baseline: 22.9501x; 22.9501x over previous
import functools

import jax
import jax.numpy as jnp
import numpy as np
from jax.experimental import pallas as pl
from jax.experimental.pallas import tpu as pltpu

_BATCH = 16384
_NODES = 1024
_PANEL = 128


def _ancestor_maps():
    parent = np.arange(_PANEL)
    parent[1:31] = (np.arange(1, 31) - 1) // 2
    maps = []
    anc = np.arange(_PANEL)
    for _ in range(4):
        anc = parent[anc]
        m = np.zeros((_PANEL, _PANEL), dtype=np.float32)
        m[anc, np.arange(_PANEL)] = 1.0
        maps.append(m)
    return np.stack(maps)


_ANC_MAPS = _ancestor_maps()


def _body(x_ref, sel_ref, o_ref):
    xb = x_ref[...]
    head = xb[:, :_PANEL]
    m = head
    sel = sel_ref[...]
    for k in range(4):
        m = jnp.minimum(
            m,
            jax.lax.dot(head, sel[k], preferred_element_type=jnp.float32),
        )
    fixed = jnp.concatenate([m, xb[:, _PANEL:]], axis=1)
    o_ref[...] = 1.0 / (1.0 + jnp.exp(-fixed))


@jax.jit
def kernel(x):
    tb = 512
    grid = _BATCH // tb
    return pl.pallas_call(
        _body,
        grid=(grid,),
        in_specs=[
            pl.BlockSpec((tb, _NODES), lambda i: (i, 0)),
            pl.BlockSpec((4, _PANEL, _PANEL), lambda i: (0, 0, 0)),
        ],
        out_specs=pl.BlockSpec((tb, _NODES), lambda i: (i, 0)),
        out_shape=jax.ShapeDtypeStruct((_BATCH, _NODES), jnp.float32),
        compiler_params=pltpu.CompilerParams(
            dimension_semantics=("parallel",),
        ),
    )(x, jnp.asarray(_ANC_MAPS))


# speedup vs baseline: 27.2893x; 1.1891x over previous
import functools

import jax
import jax.numpy as jnp
import numpy as np
from jax.experimental import pallas as pl
from jax.experimental.pallas import tpu as pltpu

_BATCH = 16384
_NODES = 1024
_PANEL = 128


def _ancestor_maps():
    parent = np.arange(_PANEL)
    parent[1:31] = (np.arange(1, 31) - 1) // 2
    maps = []
    anc = np.arange(_PANEL)
    for _ in range(4):
        anc = parent[anc]
        m = np.zeros((_PANEL, _PANEL), dtype=np.float32)
        m[anc, np.arange(_PANEL)] = 1.0
        maps.append(m)
    return np.stack(maps)


_ANC_MAPS = _ancestor_maps()


def _body(x_ref, sel_ref, o_ref):
    xb = x_ref[...]
    head = xb[:, :_PANEL]
    m = head
    sel = sel_ref[...]
    for k in range(4):
        m = jnp.minimum(
            m,
            jax.lax.dot(head, sel[k], preferred_element_type=jnp.float32),
        )
    fixed = jnp.concatenate([m, xb[:, _PANEL:]], axis=1)
    o_ref[...] = 1.0 / (1.0 + jnp.exp(-fixed))


@jax.jit
def kernel(x):
    tb = 1024
    grid = _BATCH // tb
    return pl.pallas_call(
        _body,
        grid=(grid,),
        in_specs=[
            pl.BlockSpec((tb, _NODES), lambda i: (i, 0)),
            pl.BlockSpec((4, _PANEL, _PANEL), lambda i: (0, 0, 0)),
        ],
        out_specs=pl.BlockSpec((tb, _NODES), lambda i: (i, 0)),
        out_shape=jax.ShapeDtypeStruct((_BATCH, _NODES), jnp.float32),
        compiler_params=pltpu.CompilerParams(
            dimension_semantics=("parallel",),
        ),
    )(x, jnp.asarray(_ANC_MAPS))


# speedup vs baseline: 27.9499x; 1.0242x over previous
import functools

import jax
import jax.numpy as jnp
import numpy as np
from jax.experimental import pallas as pl
from jax.experimental.pallas import tpu as pltpu

_BATCH = 16384
_NODES = 1024
_PANEL = 128


def _ancestor_maps():
    parent = np.arange(_PANEL)
    parent[1:31] = (np.arange(1, 31) - 1) // 2
    maps = []
    anc = np.arange(_PANEL)
    for _ in range(4):
        anc = parent[anc]
        m = np.zeros((_PANEL, _PANEL), dtype=np.float32)
        m[anc, np.arange(_PANEL)] = 1.0
        maps.append(m)
    return np.stack(maps)


_ANC_MAPS = _ancestor_maps()


def _body(x_ref, sel_ref, o_ref):
    xb = x_ref[...]
    head = xb[:, :_PANEL]
    m = head
    sel = sel_ref[...]
    for k in range(4):
        m = jnp.minimum(
            m,
            jax.lax.dot(head, sel[k], preferred_element_type=jnp.float32),
        )
    fixed = jnp.concatenate([m, xb[:, _PANEL:]], axis=1)
    o_ref[...] = 1.0 / (1.0 + jnp.exp(-fixed))


@jax.jit
def kernel(x):
    tb = 2048
    grid = _BATCH // tb
    return pl.pallas_call(
        _body,
        grid=(grid,),
        in_specs=[
            pl.BlockSpec((tb, _NODES), lambda i: (i, 0)),
            pl.BlockSpec((4, _PANEL, _PANEL), lambda i: (0, 0, 0)),
        ],
        out_specs=pl.BlockSpec((tb, _NODES), lambda i: (i, 0)),
        out_shape=jax.ShapeDtypeStruct((_BATCH, _NODES), jnp.float32),
        compiler_params=pltpu.CompilerParams(
            dimension_semantics=("parallel",),
        ),
    )(x, jnp.asarray(_ANC_MAPS))
